# baseline (speedup 1.0000x reference)
import functools

import jax
import jax.numpy as jnp
import numpy as np
from jax import lax
from jax.experimental import pallas as pl
from jax.experimental.pallas import tpu as pltpu
from jax.experimental.pallas import tpu_sc as plsc

B, S, D, V = 4, 2048, 1024, 128
NC, NS = 2, 16
NW = NC * NS
DH = D // 2
SPW = S // (NW // 2)
K = 8
NCHUNK = SPW // K
NGROUP = NCHUNK // 2
NPE = 2
NBUF = 2
LANES = 16
WV = DH // LANES


def _pe_table() -> np.ndarray:
    even_i = np.arange(0, D, 2, dtype=np.float32)
    denom = np.power(np.float32(10000.0), even_i / np.float32(D))
    pos = np.arange(S, dtype=np.float32).reshape(S, 1)
    even_pe = np.sin(pos / denom)
    odd_pe = np.cos(pos / denom)
    return np.stack([even_pe, odd_pe], axis=2).reshape(S, D).astype(np.float32)


_MESH = plsc.VectorSubcoreMesh(core_axis_name="c", subcore_axis_name="s")


@functools.partial(
    pl.kernel,
    out_type=jax.ShapeDtypeStruct((B, S, D), jnp.float32),
    mesh=_MESH,
    scratch_types=(
        [pltpu.VMEM((V, DH), jnp.float32)]
        + [pltpu.VMEM((B, SPW), jnp.int32)]
        + [pltpu.VMEM((K, DH), jnp.float32) for _ in range(NPE)]
        + [pltpu.VMEM((B, K, DH), jnp.float32) for _ in range(NBUF)]
        + [pltpu.SemaphoreType.DMA]
        + [pltpu.SemaphoreType.DMA for _ in range(NPE)]
        + [pltpu.SemaphoreType.DMA for _ in range(NBUF)]
    ),
)
def _embed_pe(ids_hbm, table_hbm, pe_hbm, out_hbm, *scratch):
    tab_v = scratch[0]
    ids_v = scratch[1]
    pe_bufs = scratch[2 : 2 + NPE]
    out_bufs = scratch[2 + NPE : 2 + NPE + NBUF]
    sem_in = scratch[2 + NPE + NBUF]
    sems_pe = scratch[3 + NPE + NBUF : 3 + 2 * NPE + NBUF]
    sems_o = scratch[3 + 2 * NPE + NBUF :]

    wid = lax.axis_index("s") * NC + lax.axis_index("c")
    h = wid % 2
    hoff = h * DH
    s_base = (wid // 2) * SPW

    t_cp = pltpu.async_copy(table_hbm.at[:, pl.ds(hoff, DH)], tab_v, sem_in)
    i_cp = pltpu.async_copy(ids_hbm.at[:, pl.ds(s_base, SPW)], ids_v, sem_in)

    def issue_pe(i, sub):
        return pltpu.async_copy(
            pe_hbm.at[pl.ds(s_base + i * K, K), pl.ds(hoff, DH)],
            pe_bufs[sub],
            sems_pe[sub],
        )

    def issue_out(i, sub):
        return pltpu.async_copy(
            out_bufs[sub],
            out_hbm.at[:, pl.ds(s_base + i * K, K), pl.ds(hoff, DH)],
            sems_o[sub],
        )

    def wait_pe(sub):
        pltpu.make_async_copy(
            pe_hbm.at[pl.ds(0, K), pl.ds(0, DH)], pe_bufs[sub], sems_pe[sub]
        ).wait()

    def wait_out(sub):
        pltpu.make_async_copy(
            out_bufs[sub], out_hbm.at[:, pl.ds(0, K), pl.ds(0, DH)], sems_o[sub]
        ).wait()

    t_cp.wait()
    i_cp.wait()
    issue_pe(0, 0)
    issue_pe(1, 1)
    issue_out(0, 0)
    issue_out(1, 1)

    def group(g, carry):
        vecs = [ids_v[b, pl.ds(g * 2 * K, 2 * K)] for b in range(B)]
        for sub in range(2):
            i = 2 * g + sub
            wait_pe(sub)
            wait_out(sub)
            pe_v, out_v = pe_bufs[sub], out_bufs[sub]
            rids = [[vecs[b][sub * K + j] for j in range(K)] for b in range(B)]

            def col_body(c, carry2):
                woff = c * LANES
                for j in range(K):
                    pe_vec = pe_v[j, pl.ds(woff, LANES)]
                    for b in range(B):
                        out_v[b, j, pl.ds(woff, LANES)] = (
                            tab_v[rids[b][j], pl.ds(woff, LANES)] + pe_vec
                        )
                return carry2

            lax.fori_loop(0, WV, col_body, 0)
            issue_out(i, sub)

            @pl.when(g < NGROUP - 1)
            def _():
                issue_pe(i + 2, sub)

        return carry

    lax.fori_loop(0, NGROUP, group, 0)
    wait_out(0)
    wait_out(1)


def kernel(token_ids, embedding_table):
    pe = jnp.asarray(_pe_table())
    return _embed_pe(token_ids, embedding_table, pe)


# speedup vs baseline: 1.6659x; 1.6659x over previous
import functools

import jax
import jax.numpy as jnp
import numpy as np
from jax import lax
from jax.experimental import pallas as pl
from jax.experimental.pallas import tpu as pltpu
from jax.experimental.pallas import tpu_sc as plsc

B, S, D, V = 4, 2048, 1024, 128
NC, NS = 2, 16
NW = NC * NS
SPW = S // NW
K = 8
NCHUNK = SPW // K
NBUF = 3
LANES = 16
CPR = D // LANES


def _pe_table() -> np.ndarray:
    even_i = np.arange(0, D, 2, dtype=np.float32)
    denom = np.power(np.float32(10000.0), even_i / np.float32(D))
    pos = np.arange(S, dtype=np.float32).reshape(S, 1)
    even_pe = np.sin(pos / denom)
    odd_pe = np.cos(pos / denom)
    return np.stack([even_pe, odd_pe], axis=2).reshape(S, D).astype(np.float32)


_MESH = plsc.VectorSubcoreMesh(core_axis_name="c", subcore_axis_name="s")


@functools.partial(
    pl.kernel,
    out_type=jax.ShapeDtypeStruct((B, S, D), jnp.float32),
    mesh=_MESH,
    scratch_types=(
        [pltpu.VMEM((B * K,), jnp.int32) for _ in range(NBUF)]
        + [pltpu.VMEM((B * K, D), jnp.float32) for _ in range(NBUF)]
        + [pltpu.VMEM((K, D), jnp.float32) for _ in range(NBUF)]
        + [pltpu.SemaphoreType.DMA for _ in range(1 + 2 * NBUF)]
    ),
)
def _embed_pe(ids_hbm, table_hbm, pe_hbm, out_hbm, *scratch):
    idx_bufs = scratch[0:NBUF]
    row_bufs = scratch[NBUF : 2 * NBUF]
    pe_bufs = scratch[2 * NBUF : 3 * NBUF]
    sem_idx = scratch[3 * NBUF]
    sems_in = scratch[3 * NBUF + 1 : 3 * NBUF + 1 + NBUF]
    sems_out = scratch[3 * NBUF + 1 + NBUF :]

    wid = lax.axis_index("s") * NC + lax.axis_index("c")
    s_base = wid * SPW

    def s_of(i):
        return s_base + i * K

    def issue_idx(i):
        idx_v = idx_bufs[i % NBUF]
        return [
            pltpu.async_copy(
                ids_hbm.at[b, pl.ds(s_of(i), K)], idx_v.at[pl.ds(b * K, K)], sem_idx
            )
            for b in range(B)
        ]

    def issue_in(i):
        sem = sems_in[i % NBUF]
        return (
            pltpu.async_copy(table_hbm.at[idx_bufs[i % NBUF]], row_bufs[i % NBUF], sem),
            pltpu.async_copy(pe_hbm.at[pl.ds(s_of(i), K)], pe_bufs[i % NBUF], sem),
        )

    def issue_out(i):
        rows_v, sem = row_bufs[i % NBUF], sems_out[i % NBUF]
        return [
            pltpu.async_copy(
                rows_v.at[pl.ds(b * K, K)], out_hbm.at[b, pl.ds(s_of(i), K)], sem
            )
            for b in range(B)
        ]

    def add_pe(i):
        rows_v, pe_v = row_bufs[i % NBUF], pe_bufs[i % NBUF]

        def body(c, carry):
            coff = c * LANES
            for j in range(K):
                pe_vec = pe_v[j, pl.ds(coff, LANES)]
                for b in range(B):
                    row = b * K + j
                    rows_v[row, pl.ds(coff, LANES)] = (
                        rows_v[row, pl.ds(coff, LANES)] + pe_vec
                    )
            return carry

        lax.fori_loop(0, CPR, body, 0)

    pend_idx, pend_in, pend_out = {}, {}, {}
    pend_idx[0] = issue_idx(0)
    for cp in pend_idx.pop(0):
        cp.wait()
    pend_in[0] = issue_in(0)
    if NCHUNK > 1:
        pend_idx[1] = issue_idx(1)
    for i in range(NCHUNK):
        nxt = i + 1
        if nxt < NCHUNK:
            if nxt - NBUF >= 0:
                for cp in pend_out.pop(nxt - NBUF):
                    cp.wait()
            for cp in pend_idx.pop(nxt):
                cp.wait()
            pend_in[nxt] = issue_in(nxt)
        g_cp, pe_cp = pend_in.pop(i)
        g_cp.wait()
        pe_cp.wait()
        if i + 2 < NCHUNK:
            pend_idx[i + 2] = issue_idx(i + 2)
        add_pe(i)
        pend_out[i] = issue_out(i)
    for i in sorted(pend_out):
        for cp in pend_out[i]:
            cp.wait()


def kernel(token_ids, embedding_table):
    pe = jnp.asarray(_pe_table())
    return _embed_pe(token_ids.astype(jnp.int32), embedding_table, pe)
